# baseline (speedup 1.0000x reference)
import functools

import jax
import jax.numpy as jnp
from jax import lax
from jax.experimental import pallas as pl
from jax.experimental.pallas import tpu as pltpu
from jax.experimental.pallas import tpu_sc as plsc

B = 32
T = 64
V = 32000
CLIP = 30.0

_info = plsc.get_sparse_core_info()
_NC, _NS = _info.num_cores, _info.num_subcores
_NW = _NC * _NS


def _sc_scores_body(x_hbm, tgt_hbm, scores_hbm, tgt_v, gbuf, row_v, sem):
    i = lax.axis_index("s") * _NC + lax.axis_index("c")

    pltpu.sync_copy(tgt_hbm.at[pl.ds(i * T, T)], tgt_v)

    lanes = lax.iota(jnp.int32, 16)
    tb = [tgt_v[pl.ds(c * 16, 16)] + (c * 16 + lanes) * V for c in range(T // 16)]

    dnums = lax.GatherDimensionNumbers(
        offset_dims=(), collapsed_slice_dims=(0,), start_index_map=(0,))

    def lanesum(v):
        for sh in (8, 4, 2, 1):
            p = lax.gather(v, (lanes ^ sh)[:, None], dnums, (1,),
                           unique_indices=True,
                           mode=lax.GatherScatterMode.PROMISE_IN_BOUNDS)
            v = v + p
        return v

    def j_body(j, carry):
        acc_lo, acc_hi = carry
        copies = []
        for c in range(T // 16):
            idx = tb[c] + j * (T * V)
            copies.append(
                pltpu.make_async_copy(x_hbm.at[idx], gbuf.at[pl.ds(c * 16, 16)], sem)
            )
        for cp in copies:
            cp.start()
        for cp in copies:
            cp.wait()
        v = jnp.zeros((16,), jnp.float32)
        for c in range(T // 16):
            g = gbuf[pl.ds(c * 16, 16)]
            v = v + jnp.clip(g, -CLIP, CLIP)
        s = lanesum(v)
        jl = j % 16
        in_lo = j < 16
        acc_lo = jnp.where((lanes == jl) & in_lo, s, acc_lo)
        acc_hi = jnp.where((lanes == jl) & jnp.logical_not(in_lo), s, acc_hi)
        return acc_lo, acc_hi

    zeros = jnp.zeros((16,), jnp.float32)
    acc_lo, acc_hi = lax.fori_loop(0, B, j_body, (zeros, zeros))
    row_v[pl.ds(0, 16)] = acc_lo
    row_v[pl.ds(16, 16)] = acc_hi
    pltpu.sync_copy(row_v, scores_hbm.at[i])


@functools.partial(jax.jit, static_argnames=())
def _sc_scores(x_flat, tgt_flat):
    mesh = plsc.VectorSubcoreMesh(core_axis_name="c", subcore_axis_name="s")
    k = pl.kernel(
        _sc_scores_body,
        out_type=jax.ShapeDtypeStruct((B, B), jnp.float32),
        mesh=mesh,
        scratch_types=[
            pltpu.VMEM((T,), jnp.int32),
            pltpu.VMEM((T,), jnp.float32),
            pltpu.VMEM((B,), jnp.float32),
            pltpu.SemaphoreType.DMA,
        ],
        compiler_params=pltpu.CompilerParams(needs_layout_passes=False),
    )
    return k(x_flat, tgt_flat)


def _tc_loss_body(scores_ref, dbuf_ref, out_ref):
    s = scores_ref[...]
    d = dbuf_ref[...]
    m = jnp.max(s, axis=1, keepdims=True)
    e = jnp.exp(s - m)
    lse = m + jnp.log(jnp.sum(e, axis=1, keepdims=True))
    ii = lax.broadcasted_iota(jnp.int32, (B, B), 0)
    jj = lax.broadcasted_iota(jnp.int32, (B, B), 1)
    num = jnp.sum(jnp.where(ii == jj, s, 0.0), axis=1, keepdims=True)
    mx = jnp.maximum(lse, d)
    la = mx + jnp.log1p(jnp.exp(-jnp.abs(lse - d)))
    out_ref[...] = num - la


def _tc_loss(scores, dbuf):
    return pl.pallas_call(
        _tc_loss_body,
        out_shape=jax.ShapeDtypeStruct((B, 1), jnp.float32),
    )(scores, dbuf)


def kernel(input, targets, denom, b):
    x_flat = input.reshape(B * T * V)
    tgt_flat = targets.astype(jnp.int32).reshape(B * T)
    scores = _sc_scores(x_flat, tgt_flat)
    dbuf = lax.dynamic_slice(denom, (b * B, b), (B, 1))
    return _tc_loss(scores, dbuf)
